# baseline (speedup 1.0000x reference)
import functools

import jax
import jax.numpy as jnp
from jax import lax
from jax.experimental import pallas as pl
from jax.experimental.pallas import tpu as pltpu
from jax.experimental.pallas import tpu_sc as plsc

_NB = 16
_RADIUS = 0.05
_BLOCK = 2000

_N = 100000
_NW = 32
_A = _N // _NW
_CH = 120
_NCH = 26
_STAGE = 3136


def _tc_body(f_ref, p_ref, w_ref, br_ref, out_ref):
    f = f_ref[...].astype(jnp.bfloat16)
    rel = jnp.dot(f, w_ref[...], preferred_element_type=jnp.float32)
    p = p_ref[...]
    anchor = jnp.concatenate([p] * _NB, axis=1)
    out_ref[:, : _NB * 3] = anchor + rel + br_ref[...]


def _sc_points_body(rows_ref, x_ref, inv0_ref, inv1_ref, outv_ref,
                    sem0_ref, sem1_ref):
    wid = lax.axis_index("s") * 2 + lax.axis_index("c")
    s = (wid * _A) // 8 * 8
    e = ((wid + 1) * _A) // 8 * 8
    cols = [3 * lax.iota(jnp.int32, 16) + j for j in range(3)]
    invs = [inv0_ref, inv1_ref]
    sems = [sem0_ref, sem1_ref]

    def in_copy(c, buf):
        return pltpu.make_async_copy(
            rows_ref.at[pl.ds(s + c * _CH, _CH), :], invs[buf], sems[buf])

    def process(ac, inv_ref, ch):
        def group(g, carry):
            for u in range(8):
                a = g * 8 + u
                row = jnp.zeros((16,), jnp.int32) + a
                for j in range(3):
                    v = plsc.load_gather(inv_ref, [row, cols[j]])
                    outv_ref[j, pl.ds(a * 16, 16)] = v
            return carry

        lax.fori_loop(0, ch // 8, group, 0)
        pltpu.sync_copy(outv_ref.at[:, pl.ds(0, ch * 16)],
                        x_ref.at[:, pl.ds(ac * 16, ch * 16)])

    in_copy(0, 0).start()
    in_copy(1, 1).start()

    def pair(p_idx, carry):
        for b in range(2):
            c = p_idx * 2 + b
            in_copy(c, b).wait()
            process(s + c * _CH, invs[b], _CH)

            @pl.when(c + 2 < _NCH)
            def _():
                in_copy(c + 2, b).start()
        return carry

    lax.fori_loop(0, _NCH // 2, pair, 0)

    @pl.when(e - s - _NCH * _CH == 8)
    def _():
        a0 = s + _NCH * _CH
        pltpu.sync_copy(rows_ref.at[pl.ds(a0, 8), :],
                        inv0_ref.at[pl.ds(0, 8), :])
        process(a0, inv0_ref, 8)


def _sc_batch_body(batch_ref, outb_ref, stage_ref, outbv_ref):
    wid = lax.axis_index("s") * 2 + lax.axis_index("c")
    base = wid * _A
    astart = jnp.minimum((base // 8) * 8, _N - _STAGE)
    off = base - astart
    pltpu.sync_copy(batch_ref.at[pl.ds(astart, _STAGE)], stage_ref)

    def bgroup(g, carry):
        for u in range(5):
            t = g * 5 + u
            idx = jnp.zeros((16,), jnp.int32) + (t + off)
            outbv_ref[pl.ds(t * 16, 16)] = plsc.load_gather(stage_ref, [idx])
        return carry

    lax.fori_loop(0, _A // 5, bgroup, 0)
    pltpu.sync_copy(outbv_ref, outb_ref.at[pl.ds(base * _NB, _A * _NB)])


def kernel(points, features, batch, W, b):
    n, d = features.shape
    wr = (W * _RADIUS).astype(jnp.bfloat16)
    br = (b * _RADIUS).reshape(1, _NB * 3)

    mesh = plsc.VectorSubcoreMesh(core_axis_name="c", subcore_axis_name="s")
    out_batch = functools.partial(
        pl.kernel,
        out_type=jax.ShapeDtypeStruct((n * _NB,), batch.dtype),
        mesh=mesh,
        compiler_params=pltpu.CompilerParams(needs_layout_passes=False),
        scratch_types=[
            pltpu.VMEM((_STAGE,), jnp.int32),
            pltpu.VMEM((_A * _NB,), jnp.int32),
        ],
    )(_sc_batch_body)(batch)

    rows = pl.pallas_call(
        _tc_body,
        grid=(n // _BLOCK,),
        in_specs=[
            pl.BlockSpec((_BLOCK, d), lambda i: (i, 0)),
            pl.BlockSpec((_BLOCK, 3), lambda i: (i, 0)),
            pl.BlockSpec((d, _NB * 3), lambda i: (0, 0)),
            pl.BlockSpec((1, _NB * 3), lambda i: (0, 0)),
        ],
        out_specs=pl.BlockSpec((_BLOCK, 128), lambda i: (i, 0)),
        out_shape=jax.ShapeDtypeStruct((n, 128), jnp.float32),
    )(features, points, wr, br)

    xt = functools.partial(
        pl.kernel,
        out_type=jax.ShapeDtypeStruct((3, n * _NB), jnp.float32),
        mesh=mesh,
        compiler_params=pltpu.CompilerParams(needs_layout_passes=False),
        scratch_types=[
            pltpu.VMEM((_CH, 128), jnp.float32),
            pltpu.VMEM((_CH, 128), jnp.float32),
            pltpu.VMEM((3, _CH * _NB), jnp.float32),
            pltpu.SemaphoreType.DMA,
            pltpu.SemaphoreType.DMA,
        ],
    )(_sc_points_body)(rows)

    return xt.T, out_batch


# speedup vs baseline: 1.0002x; 1.0002x over previous
import functools

import jax
import jax.numpy as jnp
from jax import lax
from jax.experimental import pallas as pl
from jax.experimental.pallas import tpu as pltpu
from jax.experimental.pallas import tpu_sc as plsc

_NB = 16
_RADIUS = 0.05
_BLOCK = 2000

_N = 100000
_H = _N // 2
_NW = 32
_A = _N // _NW
_CH = 120
_NCH = 13
_STAGE = 3136


def _tc_body(f_ref, p_ref, w_ref, br_ref, out_ref):
    f = f_ref[...].astype(jnp.bfloat16)
    rel = jnp.dot(f, w_ref[...], preferred_element_type=jnp.float32)
    p = p_ref[...]
    anchor = jnp.concatenate([p] * _NB, axis=1)
    out_ref[:, : _NB * 3] = anchor + rel + br_ref[...]


def _expand_batch(batch_ref, outb_ref, stage_ref, outbv_ref, wid):
    base = wid * _A
    astart = jnp.minimum((base // 8) * 8, _N - _STAGE)
    off = base - astart
    pltpu.sync_copy(batch_ref.at[pl.ds(astart, _STAGE)], stage_ref)

    def bgroup(g, carry):
        for u in range(5):
            t = g * 5 + u
            idx = jnp.zeros((16,), jnp.int32) + (t + off)
            outbv_ref[pl.ds(t * 16, 16)] = plsc.load_gather(stage_ref, [idx])
        return carry

    lax.fori_loop(0, _A // 5, bgroup, 0)
    pltpu.sync_copy(outbv_ref, outb_ref.at[pl.ds(base * _NB, _A * _NB)])


def _expand_points(rows_ref, x_ref, inv0_ref, inv1_ref, outv_ref,
                   sem0_ref, sem1_ref, wid):
    s = (wid * _H // _NW) // 8 * 8
    e = ((wid + 1) * _H // _NW) // 8 * 8
    cols = [3 * lax.iota(jnp.int32, 16) + j for j in range(3)]
    invs = [inv0_ref, inv1_ref]
    sems = [sem0_ref, sem1_ref]

    def in_copy(c, buf):
        return pltpu.make_async_copy(
            rows_ref.at[pl.ds(s + c * _CH, _CH), :], invs[buf], sems[buf])

    def process(ac, inv_ref, ch):
        def group(g, carry):
            for u in range(8):
                a = g * 8 + u
                row = jnp.zeros((16,), jnp.int32) + a
                for j in range(3):
                    v = plsc.load_gather(inv_ref, [row, cols[j]])
                    outv_ref[j, pl.ds(a * 16, 16)] = v
            return carry

        lax.fori_loop(0, ch // 8, group, 0)
        pltpu.sync_copy(outv_ref.at[:, pl.ds(0, ch * 16)],
                        x_ref.at[:, pl.ds(ac * 16, ch * 16)])

    in_copy(0, 0).start()
    in_copy(1, 1).start()

    def pair(p_idx, carry):
        for b in range(2):
            c = p_idx * 2 + b
            in_copy(c, b).wait()
            process(s + c * _CH, invs[b], _CH)

            @pl.when(c + 2 < _NCH)
            def _():
                in_copy(c + 2, b).start()
        return carry

    lax.fori_loop(0, _NCH // 2, pair, 0)
    in_copy(_NCH - 1, 0).wait()
    process(s + (_NCH - 1) * _CH, invs[0], _CH)

    @pl.when(e - s - _NCH * _CH == 8)
    def _():
        a0 = s + _NCH * _CH
        pltpu.sync_copy(rows_ref.at[pl.ds(a0, 8), :],
                        inv1_ref.at[pl.ds(0, 8), :])
        process(a0, inv1_ref, 8)


def _sc_half0_body(rows_ref, batch_ref, x_ref, outb_ref, inv0_ref, inv1_ref,
                   outv_ref, sem0_ref, sem1_ref, stage_ref, outbv_ref):
    wid = lax.axis_index("s") * 2 + lax.axis_index("c")
    _expand_points(rows_ref, x_ref, inv0_ref, inv1_ref, outv_ref,
                   sem0_ref, sem1_ref, wid)
    _expand_batch(batch_ref, outb_ref, stage_ref, outbv_ref, wid)


def _sc_half1_body(rows_ref, x_ref, inv0_ref, inv1_ref, outv_ref,
                   sem0_ref, sem1_ref):
    wid = lax.axis_index("s") * 2 + lax.axis_index("c")
    _expand_points(rows_ref, x_ref, inv0_ref, inv1_ref, outv_ref,
                   sem0_ref, sem1_ref, wid)


def kernel(points, features, batch, W, b):
    n, d = features.shape
    wr = (W * _RADIUS).astype(jnp.bfloat16)
    br = (b * _RADIUS).reshape(1, _NB * 3)
    mesh = plsc.VectorSubcoreMesh(core_axis_name="c", subcore_axis_name="s")

    def tc_half(h):
        return pl.pallas_call(
            _tc_body,
            grid=(_H // _BLOCK,),
            in_specs=[
                pl.BlockSpec((_BLOCK, d), lambda i, h=h: (i + h * (_H // _BLOCK), 0)),
                pl.BlockSpec((_BLOCK, 3), lambda i, h=h: (i + h * (_H // _BLOCK), 0)),
                pl.BlockSpec((d, _NB * 3), lambda i: (0, 0)),
                pl.BlockSpec((1, _NB * 3), lambda i: (0, 0)),
            ],
            out_specs=pl.BlockSpec((_BLOCK, 128), lambda i: (i, 0)),
            out_shape=jax.ShapeDtypeStruct((_H, 128), jnp.float32),
        )(features, points, wr, br)

    sc_scratch = [
        pltpu.VMEM((_CH, 128), jnp.float32),
        pltpu.VMEM((_CH, 128), jnp.float32),
        pltpu.VMEM((3, _CH * _NB), jnp.float32),
        pltpu.SemaphoreType.DMA,
        pltpu.SemaphoreType.DMA,
    ]

    rows0 = tc_half(0)
    x0, out_batch = functools.partial(
        pl.kernel,
        out_type=[
            jax.ShapeDtypeStruct((3, _H * _NB), jnp.float32),
            jax.ShapeDtypeStruct((n * _NB,), batch.dtype),
        ],
        mesh=mesh,
        compiler_params=pltpu.CompilerParams(needs_layout_passes=False),
        scratch_types=sc_scratch + [
            pltpu.VMEM((_STAGE,), jnp.int32),
            pltpu.VMEM((_A * _NB,), jnp.int32),
        ],
    )(_sc_half0_body)(rows0, batch)

    rows1 = tc_half(1)
    x1 = functools.partial(
        pl.kernel,
        out_type=jax.ShapeDtypeStruct((3, _H * _NB), jnp.float32),
        mesh=mesh,
        compiler_params=pltpu.CompilerParams(needs_layout_passes=False),
        scratch_types=sc_scratch,
    )(_sc_half1_body)(rows1)

    return jnp.concatenate([x0, x1], axis=1).T, out_batch


# speedup vs baseline: 1.4593x; 1.4590x over previous
import functools

import jax
import jax.numpy as jnp
from jax import lax
from jax.experimental import pallas as pl
from jax.experimental.pallas import tpu as pltpu
from jax.experimental.pallas import tpu_sc as plsc

_NB = 16
_RADIUS = 0.05
_BLOCK = 2000

_N = 100000
_H = _N // 2
_NW = 32
_A = _N // _NW
_CH = 120
_NCH = 13
_PSTAGE = 4704
_STAGE = 3136


def _tc_body(f_ref, w_ref, br_ref, out_ref):
    f = f_ref[...].astype(jnp.bfloat16)
    rel = jnp.dot(f, w_ref[...], preferred_element_type=jnp.float32)
    out_ref[:, : _NB * 3] = rel + br_ref[...]


def _expand_batch(batch_ref, outb_ref, stage_ref, outbv_ref, wid):
    base = wid * _A
    astart = jnp.minimum((base // 8) * 8, _N - _STAGE)
    off = base - astart
    pltpu.sync_copy(batch_ref.at[pl.ds(astart, _STAGE)], stage_ref)

    def bgroup(g, carry):
        for u in range(5):
            t = g * 5 + u
            idx = jnp.zeros((16,), jnp.int32) + (t + off)
            outbv_ref[pl.ds(t * 16, 16)] = plsc.load_gather(stage_ref, [idx])
        return carry

    lax.fori_loop(0, _A // 5, bgroup, 0)
    pltpu.sync_copy(outbv_ref, outb_ref.at[pl.ds(base * _NB, _A * _NB)])


def _expand_points(rows_ref, pts_ref, x_ref, inv0_ref, inv1_ref, outv_ref,
                   ptsv_ref, sem0_ref, sem1_ref, wid, half):
    s = (wid * _H // _NW) // 8 * 8
    e = ((wid + 1) * _H // _NW) // 8 * 8
    cols = [3 * lax.iota(jnp.int32, 16) + j for j in range(3)]
    invs = [inv0_ref, inv1_ref]
    sems = [sem0_ref, sem1_ref]

    pltpu.sync_copy(
        pts_ref.at[pl.ds(3 * (half * _H + s), _PSTAGE)], ptsv_ref)

    def in_copy(c, buf):
        return pltpu.make_async_copy(
            rows_ref.at[pl.ds(s + c * _CH, _CH), :], invs[buf], sems[buf])

    def process(ac, inv_ref, ch):
        def group(g, carry):
            for u in range(8):
                a = g * 8 + u
                row = jnp.zeros((16,), jnp.int32) + a
                pbase = jnp.zeros((16,), jnp.int32) + ((ac - s + a) * 3)
                for j in range(3):
                    v = plsc.load_gather(inv_ref, [row, cols[j]])
                    p = plsc.load_gather(ptsv_ref, [pbase + j])
                    outv_ref[j, pl.ds(a * 16, 16)] = v + p
            return carry

        lax.fori_loop(0, ch // 8, group, 0)
        pltpu.sync_copy(outv_ref.at[:, pl.ds(0, ch * 16)],
                        x_ref.at[:, pl.ds(ac * 16, ch * 16)])

    in_copy(0, 0).start()
    in_copy(1, 1).start()

    def pair(p_idx, carry):
        for b in range(2):
            c = p_idx * 2 + b
            in_copy(c, b).wait()
            process(s + c * _CH, invs[b], _CH)

            @pl.when(c + 2 < _NCH)
            def _():
                in_copy(c + 2, b).start()
        return carry

    lax.fori_loop(0, _NCH // 2, pair, 0)
    in_copy(_NCH - 1, 0).wait()
    process(s + (_NCH - 1) * _CH, invs[0], _CH)

    @pl.when(e - s - _NCH * _CH == 8)
    def _():
        a0 = s + _NCH * _CH
        pltpu.sync_copy(rows_ref.at[pl.ds(a0, 8), :],
                        inv1_ref.at[pl.ds(0, 8), :])
        process(a0, inv1_ref, 8)


def _sc_half0_body(rows_ref, pts_ref, batch_ref, x_ref, outb_ref,
                   inv0_ref, inv1_ref, outv_ref, ptsv_ref,
                   sem0_ref, sem1_ref, stage_ref, outbv_ref):
    wid = lax.axis_index("s") * 2 + lax.axis_index("c")
    _expand_points(rows_ref, pts_ref, x_ref, inv0_ref, inv1_ref, outv_ref,
                   ptsv_ref, sem0_ref, sem1_ref, wid, 0)
    _expand_batch(batch_ref, outb_ref, stage_ref, outbv_ref, wid)


def _sc_half1_body(rows_ref, pts_ref, x_ref, inv0_ref, inv1_ref, outv_ref,
                   ptsv_ref, sem0_ref, sem1_ref):
    wid = lax.axis_index("s") * 2 + lax.axis_index("c")
    _expand_points(rows_ref, pts_ref, x_ref, inv0_ref, inv1_ref, outv_ref,
                   ptsv_ref, sem0_ref, sem1_ref, wid, 1)


def kernel(points, features, batch, W, b):
    n, d = features.shape
    wr = (W * _RADIUS).astype(jnp.bfloat16)
    br = (b * _RADIUS).reshape(1, _NB * 3)
    pts_flat = points.reshape(-1)
    mesh = plsc.VectorSubcoreMesh(core_axis_name="c", subcore_axis_name="s")

    def tc_half(h):
        return pl.pallas_call(
            _tc_body,
            grid=(_H // _BLOCK,),
            in_specs=[
                pl.BlockSpec((_BLOCK, d), lambda i, h=h: (i + h * (_H // _BLOCK), 0)),
                pl.BlockSpec((d, _NB * 3), lambda i: (0, 0)),
                pl.BlockSpec((1, _NB * 3), lambda i: (0, 0)),
            ],
            out_specs=pl.BlockSpec((_BLOCK, 128), lambda i: (i, 0)),
            out_shape=jax.ShapeDtypeStruct((_H, 128), jnp.float32),
        )(features, wr, br)

    sc_scratch = [
        pltpu.VMEM((_CH, 128), jnp.float32),
        pltpu.VMEM((_CH, 128), jnp.float32),
        pltpu.VMEM((3, _CH * _NB), jnp.float32),
        pltpu.VMEM((_PSTAGE,), jnp.float32),
        pltpu.SemaphoreType.DMA,
        pltpu.SemaphoreType.DMA,
    ]

    rows0 = tc_half(0)
    x0, out_batch = functools.partial(
        pl.kernel,
        out_type=[
            jax.ShapeDtypeStruct((3, _H * _NB), jnp.float32),
            jax.ShapeDtypeStruct((n * _NB,), batch.dtype),
        ],
        mesh=mesh,
        compiler_params=pltpu.CompilerParams(needs_layout_passes=False),
        scratch_types=sc_scratch + [
            pltpu.VMEM((_STAGE,), jnp.int32),
            pltpu.VMEM((_A * _NB,), jnp.int32),
        ],
    )(_sc_half0_body)(rows0, pts_flat, batch)

    rows1 = tc_half(1)
    x1 = functools.partial(
        pl.kernel,
        out_type=jax.ShapeDtypeStruct((3, _H * _NB), jnp.float32),
        mesh=mesh,
        compiler_params=pltpu.CompilerParams(needs_layout_passes=False),
        scratch_types=sc_scratch,
    )(_sc_half1_body)(rows1, pts_flat)

    return jnp.concatenate([x0, x1], axis=1).T, out_batch


# speedup vs baseline: 1.9129x; 1.3108x over previous
import functools

import jax
import jax.numpy as jnp
from jax import lax
from jax.experimental import pallas as pl
from jax.experimental.pallas import tpu as pltpu
from jax.experimental.pallas import tpu_sc as plsc

_NB = 16
_RADIUS = 0.05
_BLOCK = 2000

_N = 100000
_H = _N // 2
_NW = 32
_A = _N // _NW
_CH = 120
_NCH = 13
_PJ = 1576
_STAGE = 3136


def _tc_body(f_ref, w_ref, br_ref, out_ref):
    f = f_ref[...].astype(jnp.bfloat16)
    rel = jnp.dot(f, w_ref[...], preferred_element_type=jnp.float32)
    out_ref[:, : _NB * 3] = rel + br_ref[...]


def _expand_batch(batch_ref, outb_ref, stage_ref, outbv_ref, wid):
    base = wid * _A
    astart = jnp.minimum((base // 8) * 8, _N - _STAGE)
    off = base - astart
    pltpu.sync_copy(batch_ref.at[pl.ds(astart, _STAGE)], stage_ref)

    def bgroup(g, carry):
        for u in range(5):
            t = g * 5 + u
            idx = jnp.zeros((16,), jnp.int32) + (t + off)
            outbv_ref[pl.ds(t * 16, 16)] = plsc.load_gather(stage_ref, [idx])
        return carry

    lax.fori_loop(0, _A // 5, bgroup, 0)
    pltpu.sync_copy(outbv_ref, outb_ref.at[pl.ds(base * _NB, _A * _NB)])


def _expand_points(rows_ref, pts_ref, x_ref, inv0_ref, inv1_ref, outv_ref,
                   ptsv_ref, sem0_ref, sem1_ref, wid, half):
    s = (wid * _H // _NW) // 8 * 8
    e = ((wid + 1) * _H // _NW) // 8 * 8
    cols = [3 * lax.iota(jnp.int32, 16) + j for j in range(3)]
    invs = [inv0_ref, inv1_ref]
    sems = [sem0_ref, sem1_ref]

    for j in range(3):
        pltpu.sync_copy(
            pts_ref.at[pl.ds(j * _N + half * _H + s, _PJ)],
            ptsv_ref.at[pl.ds(j * _PJ, _PJ)])

    def in_copy(c, buf):
        return pltpu.make_async_copy(
            rows_ref.at[pl.ds(s + c * _CH, _CH), :], invs[buf], sems[buf])

    def process(ac, inv_ref, ch):
        def group(g, carry):
            for u in range(8):
                a = g * 8 + u
                row = jnp.zeros((16,), jnp.int32) + a
                pbase = jnp.zeros((16,), jnp.int32) + (ac - s + a)
                for j in range(3):
                    v = plsc.load_gather(inv_ref, [row, cols[j]])
                    p = plsc.load_gather(ptsv_ref, [pbase + j * _PJ])
                    outv_ref[j, pl.ds(a * 16, 16)] = v + p
            return carry

        lax.fori_loop(0, ch // 8, group, 0)
        pltpu.sync_copy(outv_ref.at[:, pl.ds(0, ch * 16)],
                        x_ref.at[:, pl.ds(ac * 16, ch * 16)])

    in_copy(0, 0).start()
    in_copy(1, 1).start()

    def pair(p_idx, carry):
        for b in range(2):
            c = p_idx * 2 + b
            in_copy(c, b).wait()
            process(s + c * _CH, invs[b], _CH)

            @pl.when(c + 2 < _NCH)
            def _():
                in_copy(c + 2, b).start()
        return carry

    lax.fori_loop(0, _NCH // 2, pair, 0)
    in_copy(_NCH - 1, 0).wait()
    process(s + (_NCH - 1) * _CH, invs[0], _CH)

    @pl.when(e - s - _NCH * _CH == 8)
    def _():
        a0 = s + _NCH * _CH
        pltpu.sync_copy(rows_ref.at[pl.ds(a0, 8), :],
                        inv1_ref.at[pl.ds(0, 8), :])
        process(a0, inv1_ref, 8)


def _sc_half0_body(rows_ref, pts_ref, batch_ref, x_ref, outb_ref,
                   inv0_ref, inv1_ref, outv_ref, ptsv_ref,
                   sem0_ref, sem1_ref, stage_ref, outbv_ref):
    wid = lax.axis_index("s") * 2 + lax.axis_index("c")
    _expand_points(rows_ref, pts_ref, x_ref, inv0_ref, inv1_ref, outv_ref,
                   ptsv_ref, sem0_ref, sem1_ref, wid, 0)
    _expand_batch(batch_ref, outb_ref, stage_ref, outbv_ref, wid)


def _sc_half1_body(rows_ref, pts_ref, x_ref, inv0_ref, inv1_ref, outv_ref,
                   ptsv_ref, sem0_ref, sem1_ref):
    wid = lax.axis_index("s") * 2 + lax.axis_index("c")
    _expand_points(rows_ref, pts_ref, x_ref, inv0_ref, inv1_ref, outv_ref,
                   ptsv_ref, sem0_ref, sem1_ref, wid, 1)


def kernel(points, features, batch, W, b):
    n, d = features.shape
    wr = (W * _RADIUS).astype(jnp.bfloat16)
    br = (b * _RADIUS).reshape(1, _NB * 3)
    pts_flat = points.T.reshape(-1)
    mesh = plsc.VectorSubcoreMesh(core_axis_name="c", subcore_axis_name="s")

    def tc_half(h):
        return pl.pallas_call(
            _tc_body,
            grid=(_H // _BLOCK,),
            in_specs=[
                pl.BlockSpec((_BLOCK, d), lambda i, h=h: (i + h * (_H // _BLOCK), 0)),
                pl.BlockSpec((d, _NB * 3), lambda i: (0, 0)),
                pl.BlockSpec((1, _NB * 3), lambda i: (0, 0)),
            ],
            out_specs=pl.BlockSpec((_BLOCK, 128), lambda i: (i, 0)),
            out_shape=jax.ShapeDtypeStruct((_H, 128), jnp.float32),
        )(features, wr, br)

    sc_scratch = [
        pltpu.VMEM((_CH, 128), jnp.float32),
        pltpu.VMEM((_CH, 128), jnp.float32),
        pltpu.VMEM((3, _CH * _NB), jnp.float32),
        pltpu.VMEM((3 * _PJ,), jnp.float32),
        pltpu.SemaphoreType.DMA,
        pltpu.SemaphoreType.DMA,
    ]

    rows0 = tc_half(0)
    x0, out_batch = functools.partial(
        pl.kernel,
        out_type=[
            jax.ShapeDtypeStruct((3, _H * _NB), jnp.float32),
            jax.ShapeDtypeStruct((n * _NB,), batch.dtype),
        ],
        mesh=mesh,
        compiler_params=pltpu.CompilerParams(needs_layout_passes=False),
        scratch_types=sc_scratch + [
            pltpu.VMEM((_STAGE,), jnp.int32),
            pltpu.VMEM((_A * _NB,), jnp.int32),
        ],
    )(_sc_half0_body)(rows0, pts_flat, batch)

    rows1 = tc_half(1)
    x1 = functools.partial(
        pl.kernel,
        out_type=jax.ShapeDtypeStruct((3, _H * _NB), jnp.float32),
        mesh=mesh,
        compiler_params=pltpu.CompilerParams(needs_layout_passes=False),
        scratch_types=sc_scratch,
    )(_sc_half1_body)(rows1, pts_flat)

    return jnp.concatenate([x0, x1], axis=1).T, out_batch


# speedup vs baseline: 1.9344x; 1.0113x over previous
import functools

import jax
import jax.numpy as jnp
from jax import lax
from jax.experimental import pallas as pl
from jax.experimental.pallas import tpu as pltpu
from jax.experimental.pallas import tpu_sc as plsc

_NB = 16
_RADIUS = 0.05
_BLOCK = 2000

_N = 100000
_H = _N // 2
_NW = 32
_A = _N // _NW
_CH = 120
_NCH = 13
_PJ = 1576
_STAGE = 3136


def _tc_body(f_ref, w_ref, br_ref, out_ref):
    f = f_ref[...].astype(jnp.bfloat16)
    rel = jnp.dot(f, w_ref[...], preferred_element_type=jnp.float32)
    out_ref[:, : _NB * 3] = rel + br_ref[...]


def _expand_batch(batch_ref, outb_ref, stage_ref, outbv_ref, wid):
    base = wid * _A
    astart = jnp.minimum((base // 8) * 8, _N - _STAGE)
    off = base - astart
    pltpu.sync_copy(batch_ref.at[pl.ds(astart, _STAGE)], stage_ref)

    def bgroup(g, carry):
        for u in range(5):
            t = g * 5 + u
            idx = jnp.zeros((16,), jnp.int32) + (t + off)
            outbv_ref[pl.ds(t * 16, 16)] = plsc.load_gather(stage_ref, [idx])
        return carry

    lax.fori_loop(0, _A // 5, bgroup, 0)
    pltpu.sync_copy(outbv_ref, outb_ref.at[pl.ds(base * _NB, _A * _NB)])


def _expand_points(rows_ref, pts_ref, x_ref, inv0_ref, inv1_ref, outv_ref,
                   ptsv_ref, sem0_ref, sem1_ref, wid, half):
    s = (wid * _H // _NW) // 8 * 8
    e = ((wid + 1) * _H // _NW) // 8 * 8
    invs = [inv0_ref, inv1_ref]
    sems = [sem0_ref, sem1_ref]

    for j in range(3):
        pltpu.sync_copy(
            pts_ref.at[pl.ds(j * _N + half * _H + s, _PJ)],
            ptsv_ref.at[pl.ds(j * _PJ, _PJ)])

    def in_copy(c, buf):
        return pltpu.make_async_copy(
            rows_ref.at[pl.ds(s + c * _CH, _CH), :], invs[buf], sems[buf])

    def process(ac, inv_ref, ch):
        def group(g, carry):
            for u in range(8):
                a = g * 8 + u
                pbase = jnp.zeros((16,), jnp.int32) + (ac - s + a)
                for j in range(3):
                    v = inv_ref[a, pl.ds(j * 16, 16)]
                    p = plsc.load_gather(ptsv_ref, [pbase + j * _PJ])
                    outv_ref[j, pl.ds(a * 16, 16)] = v + p
            return carry

        lax.fori_loop(0, ch // 8, group, 0)
        pltpu.sync_copy(outv_ref.at[:, pl.ds(0, ch * 16)],
                        x_ref.at[:, pl.ds(ac * 16, ch * 16)])

    in_copy(0, 0).start()
    in_copy(1, 1).start()

    def pair(p_idx, carry):
        for b in range(2):
            c = p_idx * 2 + b
            in_copy(c, b).wait()
            process(s + c * _CH, invs[b], _CH)

            @pl.when(c + 2 < _NCH)
            def _():
                in_copy(c + 2, b).start()
        return carry

    lax.fori_loop(0, _NCH // 2, pair, 0)
    in_copy(_NCH - 1, 0).wait()
    process(s + (_NCH - 1) * _CH, invs[0], _CH)

    @pl.when(e - s - _NCH * _CH == 8)
    def _():
        a0 = s + _NCH * _CH
        pltpu.sync_copy(rows_ref.at[pl.ds(a0, 8), :],
                        inv1_ref.at[pl.ds(0, 8), :])
        process(a0, inv1_ref, 8)


def _sc_half0_body(rows_ref, pts_ref, batch_ref, x_ref, outb_ref,
                   inv0_ref, inv1_ref, outv_ref, ptsv_ref,
                   sem0_ref, sem1_ref, stage_ref, outbv_ref):
    wid = lax.axis_index("s") * 2 + lax.axis_index("c")
    _expand_points(rows_ref, pts_ref, x_ref, inv0_ref, inv1_ref, outv_ref,
                   ptsv_ref, sem0_ref, sem1_ref, wid, 0)
    _expand_batch(batch_ref, outb_ref, stage_ref, outbv_ref, wid)


def _sc_half1_body(rows_ref, pts_ref, x_ref, inv0_ref, inv1_ref, outv_ref,
                   ptsv_ref, sem0_ref, sem1_ref):
    wid = lax.axis_index("s") * 2 + lax.axis_index("c")
    _expand_points(rows_ref, pts_ref, x_ref, inv0_ref, inv1_ref, outv_ref,
                   ptsv_ref, sem0_ref, sem1_ref, wid, 1)


def kernel(points, features, batch, W, b):
    n, d = features.shape
    perm = jnp.arange(_NB * 3).reshape(_NB, 3).T.reshape(-1)
    wr = (W * _RADIUS)[:, perm].astype(jnp.bfloat16)
    br = (b * _RADIUS)[perm].reshape(1, _NB * 3)
    pts_flat = points.T.reshape(-1)
    mesh = plsc.VectorSubcoreMesh(core_axis_name="c", subcore_axis_name="s")

    def tc_half(h):
        return pl.pallas_call(
            _tc_body,
            grid=(_H // _BLOCK,),
            in_specs=[
                pl.BlockSpec((_BLOCK, d), lambda i, h=h: (i + h * (_H // _BLOCK), 0)),
                pl.BlockSpec((d, _NB * 3), lambda i: (0, 0)),
                pl.BlockSpec((1, _NB * 3), lambda i: (0, 0)),
            ],
            out_specs=pl.BlockSpec((_BLOCK, 128), lambda i: (i, 0)),
            out_shape=jax.ShapeDtypeStruct((_H, 128), jnp.float32),
        )(features, wr, br)

    sc_scratch = [
        pltpu.VMEM((_CH, 128), jnp.float32),
        pltpu.VMEM((_CH, 128), jnp.float32),
        pltpu.VMEM((3, _CH * _NB), jnp.float32),
        pltpu.VMEM((3 * _PJ,), jnp.float32),
        pltpu.SemaphoreType.DMA,
        pltpu.SemaphoreType.DMA,
    ]

    rows0 = tc_half(0)
    x0, out_batch = functools.partial(
        pl.kernel,
        out_type=[
            jax.ShapeDtypeStruct((3, _H * _NB), jnp.float32),
            jax.ShapeDtypeStruct((n * _NB,), batch.dtype),
        ],
        mesh=mesh,
        compiler_params=pltpu.CompilerParams(needs_layout_passes=False),
        scratch_types=sc_scratch + [
            pltpu.VMEM((_STAGE,), jnp.int32),
            pltpu.VMEM((_A * _NB,), jnp.int32),
        ],
    )(_sc_half0_body)(rows0, pts_flat, batch)

    rows1 = tc_half(1)
    x1 = functools.partial(
        pl.kernel,
        out_type=jax.ShapeDtypeStruct((3, _H * _NB), jnp.float32),
        mesh=mesh,
        compiler_params=pltpu.CompilerParams(needs_layout_passes=False),
        scratch_types=sc_scratch,
    )(_sc_half1_body)(rows1, pts_flat)

    return jnp.concatenate([x0, x1], axis=1).T, out_batch


# speedup vs baseline: 2.1914x; 1.1328x over previous
import functools

import jax
import jax.numpy as jnp
from jax import lax
from jax.experimental import pallas as pl
from jax.experimental.pallas import tpu as pltpu
from jax.experimental.pallas import tpu_sc as plsc

_NB = 16
_RADIUS = 0.05
_BLOCK = 2000

_N = 100000
_H = _N // 2
_NW = 32
_A = _N // _NW
_CH = 120
_NCH = 13
_PJ = 1576
_STAGE = 3136


def _tc_body(f_ref, w_ref, br_ref, out_ref):
    f = f_ref[...].astype(jnp.bfloat16)
    rel = jnp.dot(f, w_ref[...], preferred_element_type=jnp.float32)
    out_ref[:, : _NB * 3] = rel + br_ref[...]


def _expand_batch(batch_ref, outb_ref, stage_ref, outbv_ref, wid):
    base = wid * _A
    astart = jnp.minimum((base // 8) * 8, _N - _STAGE)
    off = base - astart
    pltpu.sync_copy(batch_ref.at[pl.ds(astart, _STAGE)], stage_ref)

    def bgroup(g, carry):
        for u in range(5):
            t = g * 5 + u
            idx = jnp.zeros((16,), jnp.int32) + (t + off)
            outbv_ref[pl.ds(t * 16, 16)] = plsc.load_gather(stage_ref, [idx])
        return carry

    lax.fori_loop(0, _A // 5, bgroup, 0)
    pltpu.sync_copy(outbv_ref, outb_ref.at[pl.ds(base * _NB, _A * _NB)])


def _expand_points(rows_ref, pts_ref, x_ref, inv0_ref, inv1_ref, outv_ref,
                   ptsv_ref, sem0_ref, sem1_ref, wid, half):
    s = (wid * _H // _NW) // 8 * 8
    e = ((wid + 1) * _H // _NW) // 8 * 8
    invs = [inv0_ref, inv1_ref]
    sems = [sem0_ref, sem1_ref]

    for j in range(3):
        pltpu.sync_copy(
            pts_ref.at[pl.ds(j * _N + half * _H + s, _PJ)],
            ptsv_ref.at[pl.ds(j * _PJ, _PJ)])

    def in_copy(c, buf):
        return pltpu.make_async_copy(
            rows_ref.at[pl.ds(s + c * _CH, _CH), :], invs[buf], sems[buf])

    def process(ac, inv_ref, ch):
        def group(g, carry):
            for u in range(8):
                a = g * 8 + u
                pbase = jnp.zeros((16,), jnp.int32) + (ac - s + a)
                for j in range(3):
                    v = inv_ref[a, pl.ds(j * 16, 16)]
                    p = plsc.load_gather(ptsv_ref, [pbase + j * _PJ])
                    outv_ref[j, pl.ds(a * 16, 16)] = v + p
            return carry

        lax.fori_loop(0, ch // 8, group, 0)
        pltpu.sync_copy(outv_ref.at[:, pl.ds(0, ch * 16)],
                        x_ref.at[:, pl.ds((half * _H + ac) * 16, ch * 16)])

    in_copy(0, 0).start()
    in_copy(1, 1).start()

    def pair(p_idx, carry):
        for b in range(2):
            c = p_idx * 2 + b
            in_copy(c, b).wait()
            process(s + c * _CH, invs[b], _CH)

            @pl.when(c + 2 < _NCH)
            def _():
                in_copy(c + 2, b).start()
        return carry

    lax.fori_loop(0, _NCH // 2, pair, 0)
    in_copy(_NCH - 1, 0).wait()
    process(s + (_NCH - 1) * _CH, invs[0], _CH)

    @pl.when(e - s - _NCH * _CH == 8)
    def _():
        a0 = s + _NCH * _CH
        pltpu.sync_copy(rows_ref.at[pl.ds(a0, 8), :],
                        inv1_ref.at[pl.ds(0, 8), :])
        process(a0, inv1_ref, 8)


def _sc_half0_body(rows_ref, pts_ref, batch_ref, x_ref, outb_ref,
                   inv0_ref, inv1_ref, outv_ref, ptsv_ref,
                   sem0_ref, sem1_ref, stage_ref, outbv_ref):
    wid = lax.axis_index("s") * 2 + lax.axis_index("c")
    _expand_points(rows_ref, pts_ref, x_ref, inv0_ref, inv1_ref, outv_ref,
                   ptsv_ref, sem0_ref, sem1_ref, wid, 0)
    _expand_batch(batch_ref, outb_ref, stage_ref, outbv_ref, wid)


def _sc_half1_body(rows_ref, pts_ref, x_ref, inv0_ref, inv1_ref, outv_ref,
                   ptsv_ref, sem0_ref, sem1_ref):
    wid = lax.axis_index("s") * 2 + lax.axis_index("c")
    _expand_points(rows_ref, pts_ref, x_ref, inv0_ref, inv1_ref, outv_ref,
                   ptsv_ref, sem0_ref, sem1_ref, wid, 1)


def kernel(points, features, batch, W, b):
    n, d = features.shape
    perm = jnp.arange(_NB * 3).reshape(_NB, 3).T.reshape(-1)
    wr = (W * _RADIUS)[:, perm].astype(jnp.bfloat16)
    br = (b * _RADIUS)[perm].reshape(1, _NB * 3)
    pts_flat = points.T.reshape(-1)
    mesh = plsc.VectorSubcoreMesh(core_axis_name="c", subcore_axis_name="s")

    def tc_half(h):
        return pl.pallas_call(
            _tc_body,
            grid=(_H // _BLOCK,),
            in_specs=[
                pl.BlockSpec((_BLOCK, d), lambda i, h=h: (i + h * (_H // _BLOCK), 0)),
                pl.BlockSpec((d, _NB * 3), lambda i: (0, 0)),
                pl.BlockSpec((1, _NB * 3), lambda i: (0, 0)),
            ],
            out_specs=pl.BlockSpec((_BLOCK, 128), lambda i: (i, 0)),
            out_shape=jax.ShapeDtypeStruct((_H, 128), jnp.float32),
        )(features, wr, br)

    sc_scratch = [
        pltpu.VMEM((_CH, 128), jnp.float32),
        pltpu.VMEM((_CH, 128), jnp.float32),
        pltpu.VMEM((3, _CH * _NB), jnp.float32),
        pltpu.VMEM((3 * _PJ,), jnp.float32),
        pltpu.SemaphoreType.DMA,
        pltpu.SemaphoreType.DMA,
    ]

    rows0 = tc_half(0)
    x0, out_batch = functools.partial(
        pl.kernel,
        out_type=[
            jax.ShapeDtypeStruct((3, n * _NB), jnp.float32),
            jax.ShapeDtypeStruct((n * _NB,), batch.dtype),
        ],
        mesh=mesh,
        compiler_params=pltpu.CompilerParams(needs_layout_passes=False),
        scratch_types=sc_scratch + [
            pltpu.VMEM((_STAGE,), jnp.int32),
            pltpu.VMEM((_A * _NB,), jnp.int32),
        ],
    )(_sc_half0_body)(rows0, pts_flat, batch)

    rows1 = tc_half(1)
    xref = jax.new_ref(x0)
    functools.partial(
        pl.kernel,
        out_type=(),
        mesh=mesh,
        compiler_params=pltpu.CompilerParams(needs_layout_passes=False),
        scratch_types=sc_scratch,
    )(_sc_half1_body)(rows1, pts_flat, xref)

    return xref[...].T, out_batch
